# baseline (speedup 1.0000x reference)
import functools

import jax
import jax.numpy as jnp
from jax import lax
from jax.experimental import pallas as pl
from jax.experimental.pallas import tpu as pltpu
from jax.experimental.pallas import tpu_sc as plsc

N = 10000
E = 320000
F_IN = 128
H = 16
K = 3000
NC = 2

NP = 10240
NW = 32
EB = 128
NB = 80
GRP = 8
PADE = NW * NB * EB
ROWS_PER_TILE = NP // 16
R = NP // 8


def _zero_acc_slice(zbuf, acc_sh, s):
    def _zero(i, carry):
        zbuf[i, :] = jnp.zeros((16,), jnp.float32)
        return carry

    lax.fori_loop(0, ROWS_PER_TILE, _zero, 0)
    pltpu.sync_copy(zbuf, acc_sh.at[pl.ds(s * ROWS_PER_TILE, ROWS_PER_TILE)])


def _copy_out_slice(acc_sh, out_hbm, c, s):
    off = c * NP + s * ROWS_PER_TILE
    pltpu.sync_copy(acc_sh.at[pl.ds(s * ROWS_PER_TILE, ROWS_PER_TILE)],
                    out_hbm.at[pl.ds(off, ROWS_PER_TILE)])


def _seg16_body(table_hbm, srcp_hbm, dstp_hbm, out_hbm, src2, dst2, rows,
                zbuf, acc_sh, gsem, ssem):
    c = lax.axis_index("c")
    s = lax.axis_index("s")
    wid = c * 16 + s

    pltpu.sync_copy(srcp_hbm.at[wid], src2)
    pltpu.sync_copy(dstp_hbm.at[wid], dst2)
    _zero_acc_slice(zbuf, acc_sh, s)
    plsc.subcore_barrier()

    ngroups = NB // GRP

    def fire_gathers(g, buf):
        return [pltpu.async_copy(
            table_hbm.at[src2.at[g * GRP + b]], rows.at[buf, b], gsem)
            for b in range(GRP)]

    def fire_scatters(g, buf):
        return [pltpu.async_copy(
            rows.at[buf, b], acc_sh.at[dst2.at[g * GRP + b]], ssem, add=True)
            for b in range(GRP)]

    depth = 3
    descs_g = [None] * ngroups
    descs_s = [None] * ngroups
    for g in range(min(depth - 1, ngroups)):
        descs_g[g] = fire_gathers(g, g % depth)
    for g in range(ngroups):
        buf = g % depth
        nxt = g + depth - 1
        if nxt < ngroups:
            prev_s = nxt - depth
            if prev_s >= 0:
                for d in descs_s[prev_s]:
                    d.wait()
            descs_g[nxt] = fire_gathers(nxt, nxt % depth)
        for d in descs_g[g]:
            d.wait()
        descs_s[g] = fire_scatters(g, buf)
    for g in range(max(0, ngroups - depth), ngroups):
        if descs_s[g] is not None:
            for d in descs_s[g]:
                d.wait()

    plsc.subcore_barrier()
    _copy_out_slice(acc_sh, out_hbm, c, s)


def _deg_body(srcp_hbm, dstp_hbm, out_hbm, dst2, ones_rows, zbuf, acc_sh,
              ssem):
    del srcp_hbm
    c = lax.axis_index("c")
    s = lax.axis_index("s")
    wid = c * 16 + s

    pltpu.sync_copy(dstp_hbm.at[wid], dst2)

    def _ones(i, carry):
        ones_rows[i, :] = jnp.ones((16,), jnp.float32)
        return carry

    lax.fori_loop(0, EB, _ones, 0)
    _zero_acc_slice(zbuf, acc_sh, s)
    plsc.subcore_barrier()

    descs = [pltpu.async_copy(ones_rows, acc_sh.at[dst2.at[b]], ssem,
                              add=True)
             for b in range(NB)]
    for d in descs:
        d.wait()

    plsc.subcore_barrier()
    _copy_out_slice(acc_sh, out_hbm, c, s)


_SC_MESH = plsc.VectorSubcoreMesh(core_axis_name="c", subcore_axis_name="s")

_seg16 = functools.partial(
    pl.kernel,
    out_type=jax.ShapeDtypeStruct((2 * NP, H), jnp.float32),
    mesh=_SC_MESH,
    compiler_params=pltpu.CompilerParams(use_tc_tiling_on_sc=False),
    scratch_types=[
        pltpu.VMEM((NB, EB), jnp.int32),
        pltpu.VMEM((NB, EB), jnp.int32),
        pltpu.VMEM((3, GRP, EB, H), jnp.float32),
        pltpu.VMEM((ROWS_PER_TILE, H), jnp.float32),
        pltpu.VMEM_SHARED((NP, H), jnp.float32),
        pltpu.SemaphoreType.DMA,
        pltpu.SemaphoreType.DMA,
    ],
)(_seg16_body)

_deg = functools.partial(
    pl.kernel,
    out_type=jax.ShapeDtypeStruct((2 * NP, H), jnp.float32),
    mesh=_SC_MESH,
    compiler_params=pltpu.CompilerParams(use_tc_tiling_on_sc=False),
    scratch_types=[
        pltpu.VMEM((NB, EB), jnp.int32),
        pltpu.VMEM((EB, H), jnp.float32),
        pltpu.VMEM((ROWS_PER_TILE, H), jnp.float32),
        pltpu.VMEM_SHARED((NP, H), jnp.float32),
        pltpu.SemaphoreType.DMA,
    ],
)(_deg_body)


def _tca_body(xp_ref, w1bd_ref, out_ref):
    out_ref[...] = jnp.dot(xp_ref[...], w1bd_ref[...],
                           preferred_element_type=jnp.float32)


def _tcb_body(p0a_ref, p0b_ref, xw1_ref, xs1_ref, dinv_ref, invdeg_ref):
    deg = p0a_ref[...] + p0b_ref[...] + 1.0
    dinv = lax.rsqrt(deg)
    dinv_ref[...] = dinv
    invdeg_ref[...] = 1.0 / deg
    xs1_ref[...] = xw1_ref[...] * dinv


def _tcc_body(p1a_ref, p1b_ref, xw1_ref, dinv_ref, invdeg_ref, b1_ref,
              h1_ref, t2_ref):
    accs = p1a_ref[...] + p1b_ref[...]
    h1 = jax.nn.relu(dinv_ref[...] * accs + xw1_ref[...] * invdeg_ref[...]
                     + b1_ref[...])
    h1_ref[...] = h1
    t2_ref[...] = dinv_ref[...] * h1


def _tcd_body(p2a_ref, p2b_ref, h1_ref, dinv_ref, invdeg_ref, wscbd_ref,
              w2bd_ref, gt8_ref, bsc_ref, xw2_ref, keptf_ref):
    acch = p2a_ref[...] + p2b_ref[...]
    svec = dinv_ref[...] * acch + h1_ref[...] * invdeg_ref[...]
    score = jnp.dot(svec, wscbd_ref[...], preferred_element_type=jnp.float32) \
        + bsc_ref[...]
    b = lax.bitcast_convert_type(score, jnp.int32)
    v = b ^ ((b >> 31) & jnp.int32(0x7FFFFFFF))
    u = lax.bitcast_convert_type(v ^ jnp.int32(-2147483648), jnp.uint32)
    node = lax.broadcasted_iota(jnp.int32, (R, 8), 0) * 8 \
        + lax.broadcasted_iota(jnp.int32, (R, 8), 1)
    u = jnp.where(node < N, u, jnp.uint32(0))

    def _tbit(i, t):
        t2 = t | (jnp.uint32(1) << (jnp.uint32(31) - jnp.uint32(i)))
        cnt = jnp.sum((u >= t2).astype(jnp.int32))
        return jnp.where(cnt >= K, t2, t)

    t = lax.fori_loop(0, 32, _tbit, jnp.uint32(0))
    c_gt = jnp.sum((u > t).astype(jnp.int32))
    need = K - c_gt

    def _pbit(i, p):
        p2 = p | (jnp.int32(1) << (jnp.int32(14) - i))
        cnt = jnp.sum(((u == t) & (node < p2)).astype(jnp.int32))
        return jnp.where(cnt <= need, p2, p)

    p = lax.fori_loop(0, 15, _pbit, jnp.int32(0))
    kept = (u > t) | ((u == t) & (node < p))
    keptf8 = kept.astype(jnp.float32)
    kg8 = keptf8 * jnp.tanh(score)
    kg = jnp.dot(kg8, gt8_ref[...], preferred_element_type=jnp.float32)
    xq = kg * h1_ref[...]
    xw2_ref[...] = jnp.dot(xq, w2bd_ref[...],
                           preferred_element_type=jnp.float32)
    keptf_ref[...] = jnp.dot(keptf8, gt8_ref[...],
                             preferred_element_type=jnp.float32)


def _tce_body(p3a_ref, p3b_ref, xw2_ref, xs2_ref, dinv2_ref, invdeg2_ref):
    deg2 = p3a_ref[...] + p3b_ref[...] + 1.0
    dinv2 = lax.rsqrt(deg2)
    dinv2_ref[...] = dinv2
    invdeg2_ref[...] = 1.0 / deg2
    xs2_ref[...] = xw2_ref[...] * dinv2


def _tcf_body(p4a_ref, p4b_ref, xw2_ref, dinv2_ref, invdeg2_ref, keptf_ref,
              b2_ref, lng_ref, lnb_ref, gw1bd_ref, gb1_ref, bng_ref, bnb_ref,
              gw2bd_ref, gb2_ref, fw1_ref, fb1_ref, fw2_ref, fb2_ref,
              g8_ref, gt8_ref, f64_ref, f64t_ref, fold_ref, out_ref):
    acc2 = p4a_ref[...] + p4b_ref[...]
    h2 = jax.nn.relu(dinv2_ref[...] * acc2 + xw2_ref[...] * invdeg2_ref[...]
                     + b2_ref[...])
    g8 = g8_ref[...]
    gt8 = gt8_ref[...]
    mu8 = jnp.dot(h2, g8, preferred_element_type=jnp.float32) * (1.0 / H)
    mu = jnp.dot(mu8, gt8, preferred_element_type=jnp.float32)
    d = h2 - mu
    var8 = jnp.dot(d * d, g8, preferred_element_type=jnp.float32) * (1.0 / H)
    inv8 = lax.rsqrt(var8 + 1e-5)
    inv = jnp.dot(inv8, gt8, preferred_element_type=jnp.float32)
    hn = d * inv * lng_ref[...] + lnb_ref[...]
    kf8 = jnp.dot(keptf_ref[...], g8, preferred_element_type=jnp.float32) \
        * (1.0 / H)
    g = jnp.dot(hn, gw1bd_ref[...], preferred_element_type=jnp.float32) \
        + gb1_ref[...]
    kf64 = jnp.dot(kf8, f64t_ref[...], preferred_element_type=jnp.float32)
    msum = jnp.sum(kf64 * g, axis=0, keepdims=True)
    bmu8 = jnp.dot(msum, f64_ref[...], preferred_element_type=jnp.float32) \
        * (1.0 / K)
    bmu = jnp.dot(bmu8, f64t_ref[...], preferred_element_type=jnp.float32)
    dg = g - bmu
    vsum = jnp.sum(kf64 * dg * dg, axis=0, keepdims=True)
    bvar8 = jnp.dot(vsum, f64_ref[...], preferred_element_type=jnp.float32) \
        * (1.0 / K)
    bvar = jnp.dot(bvar8, f64t_ref[...], preferred_element_type=jnp.float32)
    gn = dg / jnp.sqrt(bvar + 1e-5) * bng_ref[...] + bnb_ref[...]
    gn = jax.nn.relu(gn)
    gsc8 = jnp.dot(gn, gw2bd_ref[...], preferred_element_type=jnp.float32) \
        + gb2_ref[...]
    m = jnp.max(jnp.where(kf8 > 0.0, gsc8, -jnp.inf))
    a8 = kf8 * jnp.exp(gsc8 - m)
    z = jnp.sum(a8)
    aexp = jnp.dot(a8, gt8, preferred_element_type=jnp.float32)
    pooled128 = jnp.sum(aexp * hn, axis=0, keepdims=True) * (1.0 / z)
    pooled = jnp.dot(pooled128, fold_ref[...],
                     preferred_element_type=jnp.float32)
    h = jnp.tanh(jnp.dot(pooled, fw1_ref[...],
                         preferred_element_type=jnp.float32) + fb1_ref[...])
    logits = jnp.dot(h, fw2_ref[...], preferred_element_type=jnp.float32) \
        + fb2_ref[...]
    logits = logits - jnp.max(logits, axis=-1, keepdims=True)
    ez = jnp.exp(logits)
    out_ref[...] = ez / jnp.sum(ez, axis=-1, keepdims=True)


def _call(body, out_shapes, *args):
    return pl.pallas_call(body, out_shape=out_shapes)(*args)


def kernel(x_origin, edge_index, edge_weight, pos, params):
    f32 = jnp.float32
    sd = jax.ShapeDtypeStruct
    src = edge_index[0].astype(jnp.int32)
    dst = edge_index[1].astype(jnp.int32)
    fill = jnp.full((PADE - E,), N, jnp.int32)
    srcp = jnp.concatenate([src, fill]).reshape(NW, NB, EB)
    dstp = jnp.concatenate([dst, fill]).reshape(NW, NB, EB)

    eye8 = jnp.eye(8, dtype=f32)
    w1bd = jnp.kron(eye8, params['W1'])
    wscbd = jnp.kron(eye8, params['Wsc'])
    w2bd = jnp.kron(eye8, params['W2'])
    gw1bd = jnp.kron(eye8, params['gW1'])
    gw2bd = jnp.kron(eye8, params['gW2'])
    g8 = jnp.kron(eye8, jnp.ones((H, 1), f32))
    gt8 = jnp.kron(eye8, jnp.ones((1, H), f32))
    f64 = jnp.kron(jnp.ones((8, 1), f32), eye8)
    f64t = jnp.kron(jnp.ones((1, 8), f32), eye8)
    fold = jnp.kron(jnp.ones((8, 1), f32), jnp.eye(H, dtype=f32))
    b1t = jnp.tile(params['b1'], 8)[None]
    b2t = jnp.tile(params['b2'], 8)[None]
    lngt = jnp.tile(params['ln_g'], 8)[None]
    lnbt = jnp.tile(params['ln_b'], 8)[None]
    gb1t = jnp.tile(params['gb1'], 8)[None]
    bngt = jnp.tile(params['bn_g'], 8)[None]
    bnbt = jnp.tile(params['bn_b'], 8)[None]
    bsct = jnp.tile(params['bsc'], 8)[None]
    gb2t = jnp.tile(params['gb2'], 8)[None]

    xp = jnp.pad(x_origin, ((0, NP - N), (0, 0))).reshape(R, 8 * F_IN)

    def seg(table_p):
        part = _seg16(table_p.reshape(NP, H), srcp, dstp)
        part = part.reshape(2 * R, 128)
        return part[:R], part[R:]

    p0 = _deg(srcp, dstp).reshape(2 * R, 128)
    p0a, p0b = p0[:R], p0[R:]
    xw1 = _call(_tca_body, sd((R, 128), f32), xp, w1bd)
    xs1, dinv1, invdeg1 = _call(
        _tcb_body, [sd((R, 128), f32)] * 3, p0a, p0b, xw1)
    p1a, p1b = seg(xs1)
    h1, t2 = _call(
        _tcc_body, [sd((R, 128), f32)] * 2,
        p1a, p1b, xw1, dinv1, invdeg1, b1t)
    p2a, p2b = seg(t2)
    xw2, keptf = _call(
        _tcd_body, [sd((R, 128), f32)] * 2,
        p2a, p2b, h1, dinv1, invdeg1, wscbd, w2bd, gt8, bsct)
    p3a, p3b = seg(keptf)
    xs2, dinv2, invdeg2 = _call(
        _tce_body, [sd((R, 128), f32)] * 3, p3a, p3b, xw2)
    p4a, p4b = seg(xs2)
    out = _call(
        _tcf_body, sd((1, NC), f32),
        p4a, p4b, xw2, dinv2, invdeg2, keptf,
        b2t, lngt, lnbt, gw1bd, gb1t, bngt, bnbt, gw2bd, gb2t,
        params['fW1'], params['fb1'].reshape(1, H // 2),
        params['fW2'], params['fb2'].reshape(1, NC),
        g8, gt8, f64, f64t, fold)
    return out


# speedup vs baseline: 1.8004x; 1.8004x over previous
import functools

import jax
import jax.numpy as jnp
from jax import lax
from jax.experimental import pallas as pl
from jax.experimental.pallas import tpu as pltpu
from jax.experimental.pallas import tpu_sc as plsc

N = 10000
E = 320000
F_IN = 128
H = 16
K = 3000
NC = 2

NP = 10240
NW = 32
EB = 128
NB = 80
GRP = 8
PADE = NW * NB * EB
ROWS_PER_TILE = NP // 16
R = NP // 8


def _zero_acc_slice(zbuf, acc_sh, s):
    def _zero(i, carry):
        zbuf[i, :] = jnp.zeros((16,), jnp.float32)
        return carry

    lax.fori_loop(0, ROWS_PER_TILE, _zero, 0)
    pltpu.sync_copy(zbuf, acc_sh.at[pl.ds(s * ROWS_PER_TILE, ROWS_PER_TILE)])


def _copy_out_slice(acc_sh, out_hbm, c, s):
    off = c * NP + s * ROWS_PER_TILE
    pltpu.sync_copy(acc_sh.at[pl.ds(s * ROWS_PER_TILE, ROWS_PER_TILE)],
                    out_hbm.at[pl.ds(off, ROWS_PER_TILE)])


def _seg16_body(table_hbm, srcp_hbm, dstp_hbm, out_hbm, src2, dst2, rows,
                zbuf, acc_sh, gsem, ssem):
    c = lax.axis_index("c")
    s = lax.axis_index("s")
    wid = c * 16 + s

    pltpu.sync_copy(srcp_hbm.at[wid], src2)
    pltpu.sync_copy(dstp_hbm.at[wid], dst2)
    _zero_acc_slice(zbuf, acc_sh, s)
    plsc.subcore_barrier()

    ngroups = NB // GRP

    def fire_gathers(g, buf):
        return [pltpu.async_copy(
            table_hbm.at[src2.at[g * GRP + b]], rows.at[buf, b], gsem)
            for b in range(GRP)]

    def fire_scatters(g, buf):
        return [pltpu.async_copy(
            rows.at[buf, b], acc_sh.at[dst2.at[g * GRP + b]], ssem, add=True)
            for b in range(GRP)]

    depth = 3
    descs_g = [None] * ngroups
    descs_s = [None] * ngroups
    for g in range(min(depth - 1, ngroups)):
        descs_g[g] = fire_gathers(g, g % depth)
    for g in range(ngroups):
        buf = g % depth
        nxt = g + depth - 1
        if nxt < ngroups:
            prev_s = nxt - depth
            if prev_s >= 0:
                for d in descs_s[prev_s]:
                    d.wait()
            descs_g[nxt] = fire_gathers(nxt, nxt % depth)
        for d in descs_g[g]:
            d.wait()
        descs_s[g] = fire_scatters(g, buf)
    for g in range(max(0, ngroups - depth), ngroups):
        if descs_s[g] is not None:
            for d in descs_s[g]:
                d.wait()

    plsc.subcore_barrier()
    _copy_out_slice(acc_sh, out_hbm, c, s)


def _deg_body(srcp_hbm, dstp_hbm, out_hbm, dst2, ones_rows, zbuf, acc_sh,
              ssem):
    del srcp_hbm
    c = lax.axis_index("c")
    s = lax.axis_index("s")
    wid = c * 16 + s

    pltpu.sync_copy(dstp_hbm.at[wid], dst2)

    def _ones(i, carry):
        ones_rows[i, :] = jnp.ones((16,), jnp.float32)
        return carry

    lax.fori_loop(0, EB, _ones, 0)
    _zero_acc_slice(zbuf, acc_sh, s)
    plsc.subcore_barrier()

    descs = [pltpu.async_copy(ones_rows, acc_sh.at[dst2.at[b]], ssem,
                              add=True)
             for b in range(NB)]
    for d in descs:
        d.wait()

    plsc.subcore_barrier()
    _copy_out_slice(acc_sh, out_hbm, c, s)


_SC_MESH = plsc.VectorSubcoreMesh(core_axis_name="c", subcore_axis_name="s")

_seg16 = functools.partial(
    pl.kernel,
    out_type=jax.ShapeDtypeStruct((2 * NP, H), jnp.float32),
    mesh=_SC_MESH,
    compiler_params=pltpu.CompilerParams(use_tc_tiling_on_sc=False),
    scratch_types=[
        pltpu.VMEM((NB, EB), jnp.int32),
        pltpu.VMEM((NB, EB), jnp.int32),
        pltpu.VMEM((3, GRP, EB, H), jnp.float32),
        pltpu.VMEM((ROWS_PER_TILE, H), jnp.float32),
        pltpu.VMEM_SHARED((NP, H), jnp.float32),
        pltpu.SemaphoreType.DMA,
        pltpu.SemaphoreType.DMA,
    ],
)(_seg16_body)

_deg = functools.partial(
    pl.kernel,
    out_type=jax.ShapeDtypeStruct((2 * NP, H), jnp.float32),
    mesh=_SC_MESH,
    compiler_params=pltpu.CompilerParams(use_tc_tiling_on_sc=False),
    scratch_types=[
        pltpu.VMEM((NB, EB), jnp.int32),
        pltpu.VMEM((EB, H), jnp.float32),
        pltpu.VMEM((ROWS_PER_TILE, H), jnp.float32),
        pltpu.VMEM_SHARED((NP, H), jnp.float32),
        pltpu.SemaphoreType.DMA,
    ],
)(_deg_body)


def _tca_body(xp_ref, w1bd_ref, out_ref):
    out_ref[...] = jnp.dot(xp_ref[...], w1bd_ref[...],
                           preferred_element_type=jnp.float32)


def _tcb_body(p0a_ref, p0b_ref, xw1_ref, xs1_ref, dinv_ref, invdeg_ref):
    deg = p0a_ref[...] + p0b_ref[...] + 1.0
    dinv = lax.rsqrt(deg)
    dinv_ref[...] = dinv
    invdeg_ref[...] = 1.0 / deg
    xs1_ref[...] = xw1_ref[...] * dinv


def _tcc_body(p1a_ref, p1b_ref, xw1_ref, dinv_ref, invdeg_ref, b1_ref,
              h1_ref, t2_ref):
    accs = p1a_ref[...] + p1b_ref[...]
    h1 = jax.nn.relu(dinv_ref[...] * accs + xw1_ref[...] * invdeg_ref[...]
                     + b1_ref[...])
    h1_ref[...] = h1
    t2_ref[...] = dinv_ref[...] * h1


def _tcd_body(p2a_ref, p2b_ref, h1_ref, dinv_ref, invdeg_ref, wscbd_ref,
              w2bd_ref, gt8_ref, bsc_ref, xw2_ref, keptf_ref):
    acch = p2a_ref[...] + p2b_ref[...]
    svec = dinv_ref[...] * acch + h1_ref[...] * invdeg_ref[...]
    score = jnp.dot(svec, wscbd_ref[...], preferred_element_type=jnp.float32) \
        + bsc_ref[...]
    b = lax.bitcast_convert_type(score, jnp.int32)
    v = b ^ ((b >> 31) & jnp.int32(0x7FFFFFFF))
    u = lax.bitcast_convert_type(v ^ jnp.int32(-2147483648), jnp.uint32)
    node = lax.broadcasted_iota(jnp.int32, (R, 8), 0) * 8 \
        + lax.broadcasted_iota(jnp.int32, (R, 8), 1)
    u = jnp.where(node < N, u, jnp.uint32(0))

    def _tbit(i, t):
        t2 = t | (jnp.uint32(1) << (jnp.uint32(31) - jnp.uint32(i)))
        cnt = jnp.sum((u >= t2).astype(jnp.int32))
        return jnp.where(cnt >= K, t2, t)

    t = lax.fori_loop(0, 32, _tbit, jnp.uint32(0))
    c_gt = jnp.sum((u > t).astype(jnp.int32))
    need = K - c_gt

    def _pbit(i, p):
        p2 = p | (jnp.int32(1) << (jnp.int32(14) - i))
        cnt = jnp.sum(((u == t) & (node < p2)).astype(jnp.int32))
        return jnp.where(cnt <= need, p2, p)

    p = lax.fori_loop(0, 15, _pbit, jnp.int32(0))
    kept = (u > t) | ((u == t) & (node < p))
    keptf8 = kept.astype(jnp.float32)
    kg8 = keptf8 * jnp.tanh(score)
    kg = jnp.dot(kg8, gt8_ref[...], preferred_element_type=jnp.float32)
    xq = kg * h1_ref[...]
    xw2_ref[...] = jnp.dot(xq, w2bd_ref[...],
                           preferred_element_type=jnp.float32)
    keptf_ref[...] = jnp.dot(keptf8, gt8_ref[...],
                             preferred_element_type=jnp.float32)


def _tce_body(p3a_ref, p3b_ref, xw2_ref, xs2_ref, dinv2_ref, invdeg2_ref):
    deg2 = p3a_ref[...] + p3b_ref[...] + 1.0
    dinv2 = lax.rsqrt(deg2)
    dinv2_ref[...] = dinv2
    invdeg2_ref[...] = 1.0 / deg2
    xs2_ref[...] = xw2_ref[...] * dinv2


def _tcf_body(p4a_ref, p4b_ref, xw2_ref, dinv2_ref, invdeg2_ref, keptf_ref,
              b2_ref, lng_ref, lnb_ref, gw1bd_ref, gb1_ref, bng_ref, bnb_ref,
              gw2bd_ref, gb2_ref, fw1_ref, fb1_ref, fw2_ref, fb2_ref,
              g8_ref, gt8_ref, f64_ref, f64t_ref, fold_ref, out_ref):
    acc2 = p4a_ref[...] + p4b_ref[...]
    h2 = jax.nn.relu(dinv2_ref[...] * acc2 + xw2_ref[...] * invdeg2_ref[...]
                     + b2_ref[...])
    g8 = g8_ref[...]
    gt8 = gt8_ref[...]
    mu8 = jnp.dot(h2, g8, preferred_element_type=jnp.float32) * (1.0 / H)
    mu = jnp.dot(mu8, gt8, preferred_element_type=jnp.float32)
    d = h2 - mu
    var8 = jnp.dot(d * d, g8, preferred_element_type=jnp.float32) * (1.0 / H)
    inv8 = lax.rsqrt(var8 + 1e-5)
    inv = jnp.dot(inv8, gt8, preferred_element_type=jnp.float32)
    hn = d * inv * lng_ref[...] + lnb_ref[...]
    kf8 = jnp.dot(keptf_ref[...], g8, preferred_element_type=jnp.float32) \
        * (1.0 / H)
    g = jnp.dot(hn, gw1bd_ref[...], preferred_element_type=jnp.float32) \
        + gb1_ref[...]
    kf64 = jnp.dot(kf8, f64t_ref[...], preferred_element_type=jnp.float32)
    msum = jnp.sum(kf64 * g, axis=0, keepdims=True)
    bmu8 = jnp.dot(msum, f64_ref[...], preferred_element_type=jnp.float32) \
        * (1.0 / K)
    bmu = jnp.dot(bmu8, f64t_ref[...], preferred_element_type=jnp.float32)
    dg = g - bmu
    vsum = jnp.sum(kf64 * dg * dg, axis=0, keepdims=True)
    bvar8 = jnp.dot(vsum, f64_ref[...], preferred_element_type=jnp.float32) \
        * (1.0 / K)
    bvar = jnp.dot(bvar8, f64t_ref[...], preferred_element_type=jnp.float32)
    gn = dg / jnp.sqrt(bvar + 1e-5) * bng_ref[...] + bnb_ref[...]
    gn = jax.nn.relu(gn)
    gsc8 = jnp.dot(gn, gw2bd_ref[...], preferred_element_type=jnp.float32) \
        + gb2_ref[...]
    m = jnp.max(jnp.where(kf8 > 0.0, gsc8, -jnp.inf))
    a8 = kf8 * jnp.exp(gsc8 - m)
    z = jnp.sum(a8)
    aexp = jnp.dot(a8, gt8, preferred_element_type=jnp.float32)
    pooled128 = jnp.sum(aexp * hn, axis=0, keepdims=True) * (1.0 / z)
    pooled = jnp.dot(pooled128, fold_ref[...],
                     preferred_element_type=jnp.float32)
    h = jnp.tanh(jnp.dot(pooled, fw1_ref[...],
                         preferred_element_type=jnp.float32) + fb1_ref[...])
    logits = jnp.dot(h, fw2_ref[...], preferred_element_type=jnp.float32) \
        + fb2_ref[...]
    logits = logits - jnp.max(logits, axis=-1, keepdims=True)
    ez = jnp.exp(logits)
    out_ref[...] = ez / jnp.sum(ez, axis=-1, keepdims=True)


def _call(body, out_shapes, *args):
    return pl.pallas_call(body, out_shape=out_shapes)(*args)


def kernel(x_origin, edge_index, edge_weight, pos, params):
    f32 = jnp.float32
    sd = jax.ShapeDtypeStruct
    src = edge_index[0].astype(jnp.int32)
    dst = edge_index[1].astype(jnp.int32)
    fill = N + (jnp.arange(PADE - E, dtype=jnp.int32) % (NP - N))
    srcp = jnp.concatenate([src, fill]).reshape(NW, NB, EB)
    dstp = jnp.concatenate([dst, fill]).reshape(NW, NB, EB)

    eye8 = jnp.eye(8, dtype=f32)
    w1bd = jnp.kron(eye8, params['W1'])
    wscbd = jnp.kron(eye8, params['Wsc'])
    w2bd = jnp.kron(eye8, params['W2'])
    gw1bd = jnp.kron(eye8, params['gW1'])
    gw2bd = jnp.kron(eye8, params['gW2'])
    g8 = jnp.kron(eye8, jnp.ones((H, 1), f32))
    gt8 = jnp.kron(eye8, jnp.ones((1, H), f32))
    f64 = jnp.kron(jnp.ones((8, 1), f32), eye8)
    f64t = jnp.kron(jnp.ones((1, 8), f32), eye8)
    fold = jnp.kron(jnp.ones((8, 1), f32), jnp.eye(H, dtype=f32))
    b1t = jnp.tile(params['b1'], 8)[None]
    b2t = jnp.tile(params['b2'], 8)[None]
    lngt = jnp.tile(params['ln_g'], 8)[None]
    lnbt = jnp.tile(params['ln_b'], 8)[None]
    gb1t = jnp.tile(params['gb1'], 8)[None]
    bngt = jnp.tile(params['bn_g'], 8)[None]
    bnbt = jnp.tile(params['bn_b'], 8)[None]
    bsct = jnp.tile(params['bsc'], 8)[None]
    gb2t = jnp.tile(params['gb2'], 8)[None]

    xp = jnp.pad(x_origin, ((0, NP - N), (0, 0))).reshape(R, 8 * F_IN)

    def seg(table_p):
        part = _seg16(table_p.reshape(NP, H), srcp, dstp)
        part = part.reshape(2 * R, 128)
        return part[:R], part[R:]

    p0 = _deg(srcp, dstp).reshape(2 * R, 128)
    p0a, p0b = p0[:R], p0[R:]
    xw1 = _call(_tca_body, sd((R, 128), f32), xp, w1bd)
    xs1, dinv1, invdeg1 = _call(
        _tcb_body, [sd((R, 128), f32)] * 3, p0a, p0b, xw1)
    p1a, p1b = seg(xs1)
    h1, t2 = _call(
        _tcc_body, [sd((R, 128), f32)] * 2,
        p1a, p1b, xw1, dinv1, invdeg1, b1t)
    p2a, p2b = seg(t2)
    xw2, keptf = _call(
        _tcd_body, [sd((R, 128), f32)] * 2,
        p2a, p2b, h1, dinv1, invdeg1, wscbd, w2bd, gt8, bsct)
    p3a, p3b = seg(keptf)
    xs2, dinv2, invdeg2 = _call(
        _tce_body, [sd((R, 128), f32)] * 3, p3a, p3b, xw2)
    p4a, p4b = seg(xs2)
    out = _call(
        _tcf_body, sd((1, NC), f32),
        p4a, p4b, xw2, dinv2, invdeg2, keptf,
        b2t, lngt, lnbt, gw1bd, gb1t, bngt, bnbt, gw2bd, gb2t,
        params['fW1'], params['fb1'].reshape(1, H // 2),
        params['fW2'], params['fb2'].reshape(1, NC),
        g8, gt8, f64, f64t, fold)
    return out


# speedup vs baseline: 2.4356x; 1.3528x over previous
import functools

import jax
import jax.numpy as jnp
from jax import lax
from jax.experimental import pallas as pl
from jax.experimental.pallas import tpu as pltpu
from jax.experimental.pallas import tpu_sc as plsc

N = 10000
E = 320000
F_IN = 128
H = 16
K = 3000
NC = 2

NP = 10240
NW = 32
EB = 128
NB = 80
GRP = 8
PADE = NW * NB * EB
ROWS_PER_TILE = NP // 16
R = NP // 8


def _zero_acc_slice(zbuf, acc_sh, s):
    def _zero(i, carry):
        zbuf[i, :] = jnp.zeros((16,), jnp.float32)
        return carry

    lax.fori_loop(0, ROWS_PER_TILE, _zero, 0)
    pltpu.sync_copy(zbuf, acc_sh.at[pl.ds(s * ROWS_PER_TILE, ROWS_PER_TILE)])


def _copy_out_slice(acc_sh, out_hbm, c, s):
    off = c * NP + s * ROWS_PER_TILE
    pltpu.sync_copy(acc_sh.at[pl.ds(s * ROWS_PER_TILE, ROWS_PER_TILE)],
                    out_hbm.at[pl.ds(off, ROWS_PER_TILE)])


def _seg16_body(table_hbm, srcp_hbm, dstp_hbm, out_hbm, src2, dst2, rows,
                zbuf, acc_sh, gsem, ssem):
    c = lax.axis_index("c")
    s = lax.axis_index("s")
    wid = c * 16 + s

    pltpu.sync_copy(srcp_hbm.at[wid], src2)
    pltpu.sync_copy(dstp_hbm.at[wid], dst2)
    _zero_acc_slice(zbuf, acc_sh, s)
    plsc.subcore_barrier()

    ngroups = NB // GRP

    def fire_gathers(g, buf):
        return [pltpu.async_copy(
            table_hbm.at[src2.at[g * GRP + b]], rows.at[buf, b], gsem)
            for b in range(GRP)]

    def fire_scatters(g, buf):
        return [pltpu.async_copy(
            rows.at[buf, b], acc_sh.at[dst2.at[g * GRP + b]], ssem, add=True)
            for b in range(GRP)]

    depth = 3
    descs_g = [None] * ngroups
    descs_s = [None] * ngroups
    for g in range(min(depth - 1, ngroups)):
        descs_g[g] = fire_gathers(g, g % depth)
    for g in range(ngroups):
        buf = g % depth
        nxt = g + depth - 1
        if nxt < ngroups:
            prev_s = nxt - depth
            if prev_s >= 0:
                for d in descs_s[prev_s]:
                    d.wait()
            descs_g[nxt] = fire_gathers(nxt, nxt % depth)
        for d in descs_g[g]:
            d.wait()
        descs_s[g] = fire_scatters(g, buf)
    for g in range(max(0, ngroups - depth), ngroups):
        if descs_s[g] is not None:
            for d in descs_s[g]:
                d.wait()

    plsc.subcore_barrier()
    _copy_out_slice(acc_sh, out_hbm, c, s)


def _deg_body(srcp_hbm, dstp_hbm, out_hbm, dst2, ones_rows, zbuf, acc_sh,
              ssem):
    del srcp_hbm
    c = lax.axis_index("c")
    s = lax.axis_index("s")
    wid = c * 16 + s

    pltpu.sync_copy(dstp_hbm.at[wid], dst2)

    def _ones(i, carry):
        ones_rows[i, :] = jnp.ones((16,), jnp.float32)
        return carry

    lax.fori_loop(0, EB, _ones, 0)
    _zero_acc_slice(zbuf, acc_sh, s)
    plsc.subcore_barrier()

    descs = [pltpu.async_copy(ones_rows, acc_sh.at[dst2.at[b]], ssem,
                              add=True)
             for b in range(NB)]
    for d in descs:
        d.wait()

    plsc.subcore_barrier()
    _copy_out_slice(acc_sh, out_hbm, c, s)


_SC_MESH = plsc.VectorSubcoreMesh(core_axis_name="c", subcore_axis_name="s")

_seg16 = functools.partial(
    pl.kernel,
    out_type=jax.ShapeDtypeStruct((2 * NP, H), jnp.float32),
    mesh=_SC_MESH,
    compiler_params=pltpu.CompilerParams(use_tc_tiling_on_sc=False),
    scratch_types=[
        pltpu.VMEM((NB, EB), jnp.int32),
        pltpu.VMEM((NB, EB), jnp.int32),
        pltpu.VMEM((3, GRP, EB, H), jnp.float32),
        pltpu.VMEM((ROWS_PER_TILE, H), jnp.float32),
        pltpu.VMEM_SHARED((NP, H), jnp.float32),
        pltpu.SemaphoreType.DMA,
        pltpu.SemaphoreType.DMA,
    ],
)(_seg16_body)

_deg = functools.partial(
    pl.kernel,
    out_type=jax.ShapeDtypeStruct((2 * NP, H), jnp.float32),
    mesh=_SC_MESH,
    compiler_params=pltpu.CompilerParams(use_tc_tiling_on_sc=False),
    scratch_types=[
        pltpu.VMEM((NB, EB), jnp.int32),
        pltpu.VMEM((EB, H), jnp.float32),
        pltpu.VMEM((ROWS_PER_TILE, H), jnp.float32),
        pltpu.VMEM_SHARED((NP, H), jnp.float32),
        pltpu.SemaphoreType.DMA,
    ],
)(_deg_body)


def _tca_body(xp_ref, w1bd_ref, out_ref):
    out_ref[...] = jnp.dot(xp_ref[...], w1bd_ref[...],
                           preferred_element_type=jnp.float32)


def _tcb_body(p0_ref, xw1_ref, xs1_ref, dinv_ref, invdeg_ref):
    deg = p0_ref[pl.ds(0, R), :] + p0_ref[pl.ds(R, R), :] + 1.0
    dinv = lax.rsqrt(deg)
    dinv_ref[...] = dinv
    invdeg_ref[...] = 1.0 / deg
    xs1_ref[...] = xw1_ref[...] * dinv


def _tcc_body(p1_ref, xw1_ref, dinv_ref, invdeg_ref, b1_ref,
              h1_ref, t2_ref):
    accs = p1_ref[pl.ds(0, R), :] + p1_ref[pl.ds(R, R), :]
    h1 = jax.nn.relu(dinv_ref[...] * accs + xw1_ref[...] * invdeg_ref[...]
                     + b1_ref[...])
    h1_ref[...] = h1
    t2_ref[...] = dinv_ref[...] * h1


def _tcd_body(p2_ref, h1_ref, dinv_ref, invdeg_ref, wscbd_ref,
              w2bd_ref, gt8_ref, bsc_ref, xw2_ref, keptf_ref):
    acch = p2_ref[pl.ds(0, R), :] + p2_ref[pl.ds(R, R), :]
    svec = dinv_ref[...] * acch + h1_ref[...] * invdeg_ref[...]
    score = jnp.dot(svec, wscbd_ref[...], preferred_element_type=jnp.float32) \
        + bsc_ref[...]
    b = lax.bitcast_convert_type(score, jnp.int32)
    v = b ^ ((b >> 31) & jnp.int32(0x7FFFFFFF))
    u = lax.bitcast_convert_type(v ^ jnp.int32(-2147483648), jnp.uint32)
    node = lax.broadcasted_iota(jnp.int32, (R, 8), 0) * 8 \
        + lax.broadcasted_iota(jnp.int32, (R, 8), 1)
    u = jnp.where(node < N, u, jnp.uint32(0))

    def _tbit(i, t):
        t2 = t | (jnp.uint32(1) << (jnp.uint32(31) - jnp.uint32(i)))
        cnt = jnp.sum((u >= t2).astype(jnp.int32))
        return jnp.where(cnt >= K, t2, t)

    t = lax.fori_loop(0, 32, _tbit, jnp.uint32(0))
    c_gt = jnp.sum((u > t).astype(jnp.int32))
    need = K - c_gt

    def _pbit(i, p):
        p2 = p | (jnp.int32(1) << (jnp.int32(14) - i))
        cnt = jnp.sum(((u == t) & (node < p2)).astype(jnp.int32))
        return jnp.where(cnt <= need, p2, p)

    p = lax.fori_loop(0, 15, _pbit, jnp.int32(0))
    kept = (u > t) | ((u == t) & (node < p))
    keptf8 = kept.astype(jnp.float32)
    kg8 = keptf8 * jnp.tanh(score)
    kg = jnp.dot(kg8, gt8_ref[...], preferred_element_type=jnp.float32)
    xq = kg * h1_ref[...]
    xw2_ref[...] = jnp.dot(xq, w2bd_ref[...],
                           preferred_element_type=jnp.float32)
    keptf_ref[...] = jnp.dot(keptf8, gt8_ref[...],
                             preferred_element_type=jnp.float32)


def _tce_body(p3_ref, xw2_ref, xs2_ref, dinv2_ref, invdeg2_ref):
    deg2 = p3_ref[pl.ds(0, R), :] + p3_ref[pl.ds(R, R), :] + 1.0
    dinv2 = lax.rsqrt(deg2)
    dinv2_ref[...] = dinv2
    invdeg2_ref[...] = 1.0 / deg2
    xs2_ref[...] = xw2_ref[...] * dinv2


def _tcf_body(p4_ref, xw2_ref, dinv2_ref, invdeg2_ref, keptf_ref,
              b2_ref, lng_ref, lnb_ref, gw1bd_ref, gb1_ref, bng_ref, bnb_ref,
              gw2bd_ref, gb2_ref, fw1_ref, fb1_ref, fw2_ref, fb2_ref,
              g8_ref, gt8_ref, f64_ref, f64t_ref, fold_ref, out_ref):
    acc2 = p4_ref[pl.ds(0, R), :] + p4_ref[pl.ds(R, R), :]
    h2 = jax.nn.relu(dinv2_ref[...] * acc2 + xw2_ref[...] * invdeg2_ref[...]
                     + b2_ref[...])
    g8 = g8_ref[...]
    gt8 = gt8_ref[...]
    mu8 = jnp.dot(h2, g8, preferred_element_type=jnp.float32) * (1.0 / H)
    mu = jnp.dot(mu8, gt8, preferred_element_type=jnp.float32)
    d = h2 - mu
    var8 = jnp.dot(d * d, g8, preferred_element_type=jnp.float32) * (1.0 / H)
    inv8 = lax.rsqrt(var8 + 1e-5)
    inv = jnp.dot(inv8, gt8, preferred_element_type=jnp.float32)
    hn = d * inv * lng_ref[...] + lnb_ref[...]
    kf8 = jnp.dot(keptf_ref[...], g8, preferred_element_type=jnp.float32) \
        * (1.0 / H)
    g = jnp.dot(hn, gw1bd_ref[...], preferred_element_type=jnp.float32) \
        + gb1_ref[...]
    kf64 = jnp.dot(kf8, f64t_ref[...], preferred_element_type=jnp.float32)
    msum = jnp.sum(kf64 * g, axis=0, keepdims=True)
    bmu8 = jnp.dot(msum, f64_ref[...], preferred_element_type=jnp.float32) \
        * (1.0 / K)
    bmu = jnp.dot(bmu8, f64t_ref[...], preferred_element_type=jnp.float32)
    dg = g - bmu
    vsum = jnp.sum(kf64 * dg * dg, axis=0, keepdims=True)
    bvar8 = jnp.dot(vsum, f64_ref[...], preferred_element_type=jnp.float32) \
        * (1.0 / K)
    bvar = jnp.dot(bvar8, f64t_ref[...], preferred_element_type=jnp.float32)
    gn = dg / jnp.sqrt(bvar + 1e-5) * bng_ref[...] + bnb_ref[...]
    gn = jax.nn.relu(gn)
    gsc8 = jnp.dot(gn, gw2bd_ref[...], preferred_element_type=jnp.float32) \
        + gb2_ref[...]
    m = jnp.max(jnp.where(kf8 > 0.0, gsc8, -jnp.inf))
    a8 = kf8 * jnp.exp(gsc8 - m)
    z = jnp.sum(a8)
    aexp = jnp.dot(a8, gt8, preferred_element_type=jnp.float32)
    pooled128 = jnp.sum(aexp * hn, axis=0, keepdims=True) * (1.0 / z)
    pooled = jnp.dot(pooled128, fold_ref[...],
                     preferred_element_type=jnp.float32)
    h = jnp.tanh(jnp.dot(pooled, fw1_ref[...],
                         preferred_element_type=jnp.float32) + fb1_ref[...])
    logits = jnp.dot(h, fw2_ref[...], preferred_element_type=jnp.float32) \
        + fb2_ref[...]
    logits = logits - jnp.max(logits, axis=-1, keepdims=True)
    ez = jnp.exp(logits)
    out_ref[...] = ez / jnp.sum(ez, axis=-1, keepdims=True)


def _call(body, out_shapes, *args):
    return pl.pallas_call(body, out_shape=out_shapes)(*args)


def kernel(x_origin, edge_index, edge_weight, pos, params):
    f32 = jnp.float32
    sd = jax.ShapeDtypeStruct
    src = edge_index[0].astype(jnp.int32)
    dst = edge_index[1].astype(jnp.int32)
    fill = N + (jnp.arange(PADE - E, dtype=jnp.int32) % (NP - N))
    srcp = jnp.concatenate([src, fill]).reshape(NW, NB, EB)
    dstp = jnp.concatenate([dst, fill]).reshape(NW, NB, EB)

    eye8 = jnp.eye(8, dtype=f32)
    w1bd = jnp.kron(eye8, params['W1'])
    wscbd = jnp.kron(eye8, params['Wsc'])
    w2bd = jnp.kron(eye8, params['W2'])
    gw1bd = jnp.kron(eye8, params['gW1'])
    gw2bd = jnp.kron(eye8, params['gW2'])
    g8 = jnp.kron(eye8, jnp.ones((H, 1), f32))
    gt8 = jnp.kron(eye8, jnp.ones((1, H), f32))
    f64 = jnp.kron(jnp.ones((8, 1), f32), eye8)
    f64t = jnp.kron(jnp.ones((1, 8), f32), eye8)
    fold = jnp.kron(jnp.ones((8, 1), f32), jnp.eye(H, dtype=f32))
    b1t = jnp.tile(params['b1'], 8)[None]
    b2t = jnp.tile(params['b2'], 8)[None]
    lngt = jnp.tile(params['ln_g'], 8)[None]
    lnbt = jnp.tile(params['ln_b'], 8)[None]
    gb1t = jnp.tile(params['gb1'], 8)[None]
    bngt = jnp.tile(params['bn_g'], 8)[None]
    bnbt = jnp.tile(params['bn_b'], 8)[None]
    bsct = jnp.tile(params['bsc'], 8)[None]
    gb2t = jnp.tile(params['gb2'], 8)[None]

    xp = jnp.pad(x_origin, ((0, NP - N), (0, 0))).reshape(R, 8 * F_IN)

    def seg(table_p):
        return _seg16(table_p.reshape(NP, H), srcp, dstp).reshape(2 * R, 128)

    p0 = _deg(srcp, dstp).reshape(2 * R, 128)
    xw1 = _call(_tca_body, sd((R, 128), f32), xp, w1bd)
    xs1, dinv1, invdeg1 = _call(
        _tcb_body, [sd((R, 128), f32)] * 3, p0, xw1)
    p1 = seg(xs1)
    h1, t2 = _call(
        _tcc_body, [sd((R, 128), f32)] * 2,
        p1, xw1, dinv1, invdeg1, b1t)
    p2 = seg(t2)
    xw2, keptf = _call(
        _tcd_body, [sd((R, 128), f32)] * 2,
        p2, h1, dinv1, invdeg1, wscbd, w2bd, gt8, bsct)
    p3 = seg(keptf)
    xs2, dinv2, invdeg2 = _call(
        _tce_body, [sd((R, 128), f32)] * 3, p3, xw2)
    p4 = seg(xs2)
    out = _call(
        _tcf_body, sd((1, NC), f32),
        p4, xw2, dinv2, invdeg2, keptf,
        b2t, lngt, lnbt, gw1bd, gb1t, bngt, bnbt, gw2bd, gb2t,
        params['fW1'], params['fb1'].reshape(1, H // 2),
        params['fW2'], params['fb2'].reshape(1, NC),
        g8, gt8, f64, f64t, fold)
    return out
